# baseline (speedup 1.0000x reference)
import functools

import jax
import jax.numpy as jnp
from jax import lax
from jax.experimental import pallas as pl
from jax.experimental.pallas import tpu as pltpu
from jax.experimental.pallas import tpu_sc as plsc

VOCAB = 1000000
EMB = 64
BATCH = 4096
SEQ = 200
NTOK = BATCH * SEQ
NC, NS, L = 2, 16, 16
NW = NC * NS
CHUNK = 64
CPW = NTOK // (NW * CHUNK)
NBUF = 5
XROWS = CPW
BR = 2000

_mesh = plsc.VectorSubcoreMesh(
    core_axis_name="c", subcore_axis_name="s", num_cores=NC, num_subcores=NS
)


def _compact_body(w_ref, o_ref):
    w = w_ref[...].reshape(BR // 2, 2, EMB)
    o_ref[...] = jnp.concatenate([w[:, 0, :], w[:, 1, :]], axis=1)


_compact = pl.pallas_call(
    _compact_body,
    out_shape=jax.ShapeDtypeStruct((VOCAB // 2, 2 * EMB), jnp.float32),
    grid=(VOCAB // BR,),
    in_specs=[pl.BlockSpec((BR, EMB), lambda i: (i, 0))],
    out_specs=pl.BlockSpec((BR // 2, 2 * EMB), lambda i: (i, 0)),
)


@functools.partial(
    pl.kernel,
    out_type=jax.ShapeDtypeStruct((NTOK, EMB), jnp.float32),
    mesh=_mesh,
    scratch_types=(
        tuple(pltpu.VMEM((1, CHUNK), jnp.int32) for _ in range(2 * NBUF)),
        tuple(pltpu.VMEM((CHUNK,), jnp.int32) for _ in range(NBUF)),
        tuple(pltpu.VMEM((CHUNK, 2 * EMB), jnp.float32) for _ in range(NBUF)),
        tuple(pltpu.VMEM((CHUNK, EMB), jnp.float32) for _ in range(NBUF)),
        tuple(pltpu.SemaphoreType.DMA for _ in range(NBUF)),
        tuple(pltpu.SemaphoreType.DMA for _ in range(NBUF)),
        tuple(pltpu.SemaphoreType.DMA for _ in range(2 * NBUF)),
    ),
    compiler_params=pltpu.CompilerParams(needs_layout_passes=False),
)
def _emb_lookup(
    x_hbm, w2_hbm, emb_hbm,
    ibuf, pidx, wide, cbuf, gsems, wsems, isems,
):
    wid = lax.axis_index("s") * NC + lax.axis_index("c")
    row0 = wid * XROWS
    x2_hbm = x_hbm

    def start_iload(j, slot):
        pltpu.async_copy(x2_hbm.at[pl.ds(row0 + j, 1)],
                         ibuf[slot], isems[slot])

    def wait_iload(j, slot):
        pltpu.make_async_copy(x2_hbm.at[pl.ds(row0 + j, 1)],
                              ibuf[slot], isems[slot]).wait()

    def set_pidx(slot, b):
        for k in range(CHUNK // L):
            v = ibuf[slot][0, pl.ds(k * L, L)]
            pidx[b][pl.ds(k * L, L)] = jax.lax.shift_right_logical(v, 1)

    def start_gather(b):
        pltpu.async_copy(w2_hbm.at[pidx[b]], wide[b], gsems[b])

    def wait_gather(b):
        pltpu.make_async_copy(w2_hbm.at[pidx[b]], wide[b], gsems[b]).wait()

    def extract(slot, b):
        @pl.loop(0, CHUNK // L)
        def _grp(g):
            v = ibuf[slot][0, pl.ds(g * L, L)]
            par = jax.lax.bitwise_and(v, 1)

            @pl.loop(0, L // 4)
            def _row(rq):
                for u in range(4):
                    rl = rq * 4 + u
                    r = g * L + rl
                    cond = jnp.take(par, jnp.full((L,), rl, jnp.int32)) == 1
                    for c in range(EMB // L):
                        lo = wide[b][r, pl.ds(c * L, L)]
                        hi = wide[b][r, pl.ds(EMB + c * L, L)]
                        cbuf[b][r, pl.ds(c * L, L)] = jnp.where(cond, hi, lo)

    def start_writeout(j, b):
        pltpu.async_copy(
            cbuf[b], emb_hbm.at[pl.ds((row0 + j) * CHUNK, CHUNK)], wsems[b]
        )

    def wait_writeout(j, b):
        pltpu.make_async_copy(
            cbuf[b], emb_hbm.at[pl.ds((row0 + j) * CHUNK, CHUNK)], wsems[b]
        ).wait()

    def do_round(r, p, *, wwait, nxt, iload):
        for b in range(NBUF):
            j = r * NBUF + b
            slot = b + p * NBUF
            other = b + (1 - p) * NBUF
            wait_gather(b)
            if wwait:
                wait_writeout(j - NBUF, b)
            extract(slot, b)
            start_writeout(j, b)
            if nxt:
                wait_iload(j + NBUF, other)
                set_pidx(other, b)
                start_gather(b)
            if iload:
                start_iload(j + 2 * NBUF, slot)

    for j in range(2 * NBUF):
        start_iload(j, j)
    for b in range(NBUF):
        wait_iload(b, b)
        set_pidx(b, b)
        start_gather(b)

    do_round(0, 0, wwait=False, nxt=True, iload=True)

    @pl.loop(0, (CPW // NBUF - 4) // 2)
    def _ring(k):
        do_round(2 * k + 1, 1, wwait=True, nxt=True, iload=True)
        do_round(2 * k + 2, 0, wwait=True, nxt=True, iload=True)

    do_round(CPW // NBUF - 3, 1, wwait=True, nxt=True, iload=True)
    do_round(CPW // NBUF - 2, 0, wwait=True, nxt=True, iload=False)
    do_round(CPW // NBUF - 1, 1, wwait=True, nxt=False, iload=False)

    for b in range(NBUF):
        wait_writeout(CPW - NBUF + b, b)


def _mask_body(x_ref, o_ref):
    o_ref[...] = jnp.where(x_ref[...] != 0, 1.0, 0.0).astype(jnp.float32)


_mask_tc = pl.pallas_call(
    _mask_body,
    out_shape=jax.ShapeDtypeStruct((BATCH, SEQ), jnp.float32),
    grid=(8,),
    in_specs=[pl.BlockSpec((BATCH // 8, SEQ), lambda i: (i, 0))],
    out_specs=pl.BlockSpec((BATCH // 8, SEQ), lambda i: (i, 0)),
)


def kernel(x, weight):
    xf = x.reshape(NTOK // CHUNK, CHUNK)
    w2 = weight.reshape(VOCAB // 2, 2 * EMB)
    emb = _emb_lookup(xf, w2)
    mask = _mask_tc(x)
    return emb.reshape(BATCH, SEQ, EMB), mask


# speedup vs baseline: 1.0363x; 1.0363x over previous
import functools

import jax
import jax.numpy as jnp
from jax import lax
from jax.experimental import pallas as pl
from jax.experimental.pallas import tpu as pltpu
from jax.experimental.pallas import tpu_sc as plsc

VOCAB = 1000000
EMB = 64
BATCH = 4096
SEQ = 200
NTOK = BATCH * SEQ
NC, NS, L = 2, 16, 16
NW = NC * NS
CHUNK = 128
CPW = NTOK // (NW * CHUNK)
NBUF = 8

_mesh = plsc.VectorSubcoreMesh(
    core_axis_name="c", subcore_axis_name="s", num_cores=NC, num_subcores=NS
)


@functools.partial(
    pl.kernel,
    out_type=(
        jax.ShapeDtypeStruct((NTOK, EMB), jnp.float32),
        jax.ShapeDtypeStruct((NTOK // CHUNK, CHUNK), jnp.float32),
    ),
    mesh=_mesh,
    scratch_types=(
        pltpu.VMEM((CPW, CHUNK), jnp.int32),
        pltpu.VMEM((CPW, CHUNK), jnp.float32),
        tuple(pltpu.VMEM((CHUNK, EMB), jnp.float32) for _ in range(NBUF)),
        tuple(pltpu.SemaphoreType.DMA for _ in range(NBUF)),
        pltpu.SemaphoreType.DMA,
    ),
    compiler_params=pltpu.CompilerParams(use_tc_tiling_on_sc=False),
)
def _emb_lookup(
    x_hbm, w_hbm, emb_hbm, mask_hbm, idx_v, mask_v, rows, gsems, msem
):
    wid = lax.axis_index("s") * NC + lax.axis_index("c")
    row0 = wid * CPW
    pltpu.sync_copy(x_hbm.at[pl.ds(row0, CPW)], idx_v)

    def start_gather(j, b):
        pltpu.async_copy(w_hbm.at[idx_v.at[j]], rows[b], gsems[b])

    def drain(j, b):
        pltpu.make_async_copy(w_hbm.at[idx_v.at[j]], rows[b], gsems[b]).wait()
        pltpu.sync_copy(rows[b], emb_hbm.at[pl.ds((row0 + j) * CHUNK, CHUNK)])

    for b in range(NBUF):
        start_gather(b, b)

    @pl.loop(0, CPW)
    def _mask(j):
        for k in range(CHUNK // L):
            v = idx_v[j, pl.ds(k * L, L)]
            mask_v[j, pl.ds(k * L, L)] = jnp.where(v != 0, 1.0, 0.0).astype(
                jnp.float32
            )

    pltpu.async_copy(mask_v, mask_hbm.at[pl.ds(row0, CPW)], msem)

    @pl.loop(0, CPW // NBUF - 1)
    def _ring(r):
        for b in range(NBUF):
            j = r * NBUF + b
            drain(j, b)
            start_gather(j + NBUF, b)

    for b in range(NBUF):
        drain(CPW - NBUF + b, b)

    pltpu.make_async_copy(mask_v, mask_hbm.at[pl.ds(row0, CPW)], msem).wait()


def kernel(x, weight):
    xf = x.reshape(NTOK // CHUNK, CHUNK)
    emb, mask = _emb_lookup(xf, weight)
    return emb.reshape(BATCH, SEQ, EMB), mask.reshape(BATCH, SEQ)
